# baseline (speedup 1.0000x reference)
import functools

import jax
import jax.numpy as jnp
from jax import lax
from jax.experimental import pallas as pl
from jax.experimental.pallas import tpu as pltpu
from jax.experimental.pallas import tpu_sc as plsc

N = 10000
E = 320000
D = 128

NC = 2
NS = 16
NW = NC * NS
EPW = E // NW
CH = 125
NCHUNK = EPW // CH
ZR = 40

DSTEP = 624
DSIZE = 640

RB = 1000
GRID = N // RB

_mesh = plsc.VectorSubcoreMesh(core_axis_name="c", subcore_axis_name="s")


GRP = 16
NGRP = NCHUNK // GRP
GP = GRP // 2


@functools.partial(
    pl.kernel,
    out_type=jax.ShapeDtypeStruct((NC, N, D), jnp.float32),
    mesh=_mesh,
    scratch_types=[
        pltpu.VMEM((GRP, CH), jnp.int32),
        pltpu.VMEM((GRP, CH), jnp.int32),
        pltpu.VMEM((CH, D), jnp.float32),
        pltpu.VMEM((CH, D), jnp.float32),
        pltpu.VMEM((ZR, D), jnp.float32),
        pltpu.VMEM_SHARED((N, D), jnp.float32),
        pltpu.SemaphoreType.DMA,
        pltpu.SemaphoreType.DMA,
        pltpu.SemaphoreType.DMA,
        pltpu.SemaphoreType.DMA,
    ],
)
def _g(t_hbm, src_hbm, dst_hbm, out_hbm,
       src_g, dst_g, rows_a, rows_b, zero_v, acc, sem_a, sem_b, sem_sa, sem_sb):
    cid = lax.axis_index("c")
    sid = lax.axis_index("s")
    wid = cid * NS + sid

    def zfill(i, c):
        zero_v[i // 8, pl.ds((i % 8) * 16, 16)] = jnp.zeros((16,), jnp.float32)
        return c

    lax.fori_loop(0, ZR * 8, zfill, 0)

    def zcopy(i, c):
        pltpu.sync_copy(zero_v, acc.at[pl.ds(sid * DSTEP + i * ZR, ZR)])
        return c

    lax.fori_loop(0, DSIZE // ZR, zcopy, 0)
    plsc.subcore_barrier()

    def group(g, c):
        gb = g * GRP
        pltpu.sync_copy(src_hbm.at[wid, pl.ds(gb, GRP)], src_g)
        pltpu.sync_copy(dst_hbm.at[wid, pl.ds(gb, GRP)], dst_g)
        pltpu.async_copy(t_hbm.at[src_g.at[0]], rows_a, sem_a)
        pltpu.async_copy(t_hbm.at[src_g.at[1]], rows_b, sem_b)

        def pair(p, c2):
            j = 2 * p
            pltpu.make_async_copy(t_hbm.at[src_g.at[j]], rows_a, sem_a).wait()
            d_sa = pltpu.async_copy(rows_a, acc.at[dst_g.at[j]], sem_sa,
                                    add=True)
            pltpu.make_async_copy(
                t_hbm.at[src_g.at[j + 1]], rows_b, sem_b).wait()
            d_sb = pltpu.async_copy(rows_b, acc.at[dst_g.at[j + 1]], sem_sb,
                                    add=True)
            d_sa.wait()

            @pl.when(p < GP - 1)
            def _():
                pltpu.async_copy(t_hbm.at[src_g.at[j + 2]], rows_a, sem_a)

            d_sb.wait()

            @pl.when(p < GP - 1)
            def _():
                pltpu.async_copy(t_hbm.at[src_g.at[j + 3]], rows_b, sem_b)

            return c2

        lax.fori_loop(0, GP, pair, 0)
        return c

    lax.fori_loop(0, NGRP, group, 0)
    plsc.subcore_barrier()
    pltpu.sync_copy(acc.at[pl.ds(sid * DSTEP, DSIZE)],
                    out_hbm.at[cid, pl.ds(sid * DSTEP, DSIZE)])


@functools.partial(
    pl.kernel,
    out_type=jax.ShapeDtypeStruct((NC, N, D), jnp.float32),
    mesh=_mesh,
    scratch_types=[
        pltpu.VMEM((NCHUNK, CH), jnp.int32),
        pltpu.VMEM((CH, D), jnp.float32),
        pltpu.VMEM((ZR, D), jnp.float32),
        pltpu.VMEM_SHARED((N, D), jnp.float32),
    ],
)
def _deg(src_hbm, out_hbm, src_v, ones_v, zero_v, acc):
    cid = lax.axis_index("c")
    sid = lax.axis_index("s")
    wid = cid * NS + sid

    e0 = jnp.where(lax.iota(jnp.int32, 16) == 0, 1.0, 0.0).astype(jnp.float32)
    z16 = jnp.zeros((16,), jnp.float32)

    def fill(i, c):
        r = i // 8
        k = i % 8
        ones_v[r, pl.ds(k * 16, 16)] = jnp.where(k == 0, e0, z16)
        return c

    lax.fori_loop(0, CH * 8, fill, 0)

    def zfill(i, c):
        zero_v[i // 8, pl.ds((i % 8) * 16, 16)] = z16
        return c

    lax.fori_loop(0, ZR * 8, zfill, 0)

    def zcopy(i, c):
        pltpu.sync_copy(zero_v, acc.at[pl.ds(sid * DSTEP + i * ZR, ZR)])
        return c

    lax.fori_loop(0, DSIZE // ZR, zcopy, 0)
    plsc.subcore_barrier()

    pltpu.sync_copy(src_hbm.at[wid], src_v)

    def body(j, c):
        pltpu.sync_copy(ones_v, acc.at[src_v.at[j]], add=True)
        return c

    lax.fori_loop(0, NCHUNK, body, 0)
    plsc.subcore_barrier()
    pltpu.sync_copy(acc.at[pl.ds(sid * DSTEP, DSIZE)],
                    out_hbm.at[cid, pl.ds(sid * DSTEP, DSIZE)])


def _tc_scale_body(deg_ref, x_ref, dis_ref, xs_ref):
    d = deg_ref[0] + deg_ref[1]
    dis = jnp.where(d > 0.0, lax.rsqrt(jnp.where(d > 0.0, d, 1.0)), 0.0)
    dis_ref[...] = dis
    xs_ref[...] = x_ref[...] * dis


def _tc_scale(deg2, x):
    return pl.pallas_call(
        _tc_scale_body,
        grid=(GRID,),
        in_specs=[
            pl.BlockSpec((NC, RB, 1), lambda i: (0, i, 0)),
            pl.BlockSpec((RB, D), lambda i: (i, 0)),
        ],
        out_specs=[
            pl.BlockSpec((RB, 1), lambda i: (i, 0)),
            pl.BlockSpec((RB, D), lambda i: (i, 0)),
        ],
        out_shape=[
            jax.ShapeDtypeStruct((N, 1), jnp.float32),
            jax.ShapeDtypeStruct((N, D), jnp.float32),
        ],
    )(deg2, x)


def _tc_mid1_body(u_ref, dis_ref, t_ref, w_ref, outp_ref, y_ref):
    dis = dis_ref[...]
    tx1 = -(dis * (u_ref[0] + u_ref[1]))
    outp_ref[...] = (
        jnp.dot(t_ref[...], w_ref[0], preferred_element_type=jnp.float32)
        + jnp.dot(tx1, w_ref[1], preferred_element_type=jnp.float32))
    y_ref[...] = dis * tx1


def _tc_mid1(u2, dis, t, w):
    return pl.pallas_call(
        _tc_mid1_body,
        grid=(GRID,),
        in_specs=[
            pl.BlockSpec((NC, RB, D), lambda i: (0, i, 0)),
            pl.BlockSpec((RB, 1), lambda i: (i, 0)),
            pl.BlockSpec((RB, D), lambda i: (i, 0)),
            pl.BlockSpec((3, D, D), lambda i: (0, 0, 0)),
        ],
        out_specs=[
            pl.BlockSpec((RB, D), lambda i: (i, 0)),
            pl.BlockSpec((RB, D), lambda i: (i, 0)),
        ],
        out_shape=[
            jax.ShapeDtypeStruct((N, D), jnp.float32),
            jax.ShapeDtypeStruct((N, D), jnp.float32),
        ],
    )(u2, dis, t, w)


def _tc_mid2_body(u_ref, dis_ref, t_ref, outp_ref, w_ref, b_ref, h_ref, hs_ref):
    dis = dis_ref[...]
    tx2 = -2.0 * (dis * (u_ref[0] + u_ref[1])) - t_ref[...]
    h = jnp.maximum(
        outp_ref[...]
        + jnp.dot(tx2, w_ref[2], preferred_element_type=jnp.float32)
        + b_ref[...], 0.0)
    h_ref[...] = h
    hs_ref[...] = dis * h


def _tc_mid2(u2, dis, t, outp, w, b):
    return pl.pallas_call(
        _tc_mid2_body,
        grid=(GRID,),
        in_specs=[
            pl.BlockSpec((NC, RB, D), lambda i: (0, i, 0)),
            pl.BlockSpec((RB, 1), lambda i: (i, 0)),
            pl.BlockSpec((RB, D), lambda i: (i, 0)),
            pl.BlockSpec((RB, D), lambda i: (i, 0)),
            pl.BlockSpec((3, D, D), lambda i: (0, 0, 0)),
            pl.BlockSpec((D,), lambda i: (0,)),
        ],
        out_specs=[
            pl.BlockSpec((RB, D), lambda i: (i, 0)),
            pl.BlockSpec((RB, D), lambda i: (i, 0)),
        ],
        out_shape=[
            jax.ShapeDtypeStruct((N, D), jnp.float32),
            jax.ShapeDtypeStruct((N, D), jnp.float32),
        ],
    )(u2, dis, t, outp, w, b)


def _tc_final_body(u_ref, dis_ref, t_ref, outp_ref, w_ref, b_ref, o_ref):
    dis = dis_ref[...]
    tx2 = -2.0 * (dis * (u_ref[0] + u_ref[1])) - t_ref[...]
    o_ref[...] = jnp.maximum(
        outp_ref[...]
        + jnp.dot(tx2, w_ref[2], preferred_element_type=jnp.float32)
        + b_ref[...], 0.0)


def _tc_final(u2, dis, t, outp, w, b):
    return pl.pallas_call(
        _tc_final_body,
        grid=(GRID,),
        in_specs=[
            pl.BlockSpec((NC, RB, D), lambda i: (0, i, 0)),
            pl.BlockSpec((RB, 1), lambda i: (i, 0)),
            pl.BlockSpec((RB, D), lambda i: (i, 0)),
            pl.BlockSpec((RB, D), lambda i: (i, 0)),
            pl.BlockSpec((3, D, D), lambda i: (0, 0, 0)),
            pl.BlockSpec((D,), lambda i: (0,)),
        ],
        out_specs=pl.BlockSpec((RB, D), lambda i: (i, 0)),
        out_shape=jax.ShapeDtypeStruct((N, D), jnp.float32),
    )(u2, dis, t, outp, w, b)


def kernel(x, edge_index, W1, b1, W2, b2):
    src_r = edge_index[0].reshape(NW, NCHUNK, CH)
    dst_r = edge_index[1].reshape(NW, NCHUNK, CH)

    deg2 = jax.lax.slice(_deg(src_r), (0, 0, 0), (NC, N, 1))
    dis, xs = _tc_scale(deg2, x)

    u1 = _g(xs, src_r, dst_r)
    outp, y1 = _tc_mid1(u1, dis, x, W1)
    u2 = _g(y1, src_r, dst_r)
    h, hs = _tc_mid2(u2, dis, x, outp, W1, b1)

    u3 = _g(hs, src_r, dst_r)
    outp2, y2 = _tc_mid1(u3, dis, h, W2)
    u4 = _g(y2, src_r, dst_r)
    return _tc_final(u4, dis, h, outp2, W2, b2)


# speedup vs baseline: 1.1145x; 1.1145x over previous
import functools

import jax
import jax.numpy as jnp
from jax import lax
from jax.experimental import pallas as pl
from jax.experimental.pallas import tpu as pltpu
from jax.experimental.pallas import tpu_sc as plsc

N = 10000
E = 320000
D = 128

NC = 2
NS = 16
NW = NC * NS
EPW = E // NW
CH = 125
NCHUNK = EPW // CH
ZR = 40

DSTEP = 624
DSIZE = 640

RB = 1000
GRID = N // RB

_mesh = plsc.VectorSubcoreMesh(core_axis_name="c", subcore_axis_name="s")


GRP = 40
NGRP = NCHUNK // GRP
GP = GRP // 2


@functools.partial(
    pl.kernel,
    out_type=jax.ShapeDtypeStruct((NC, N, D), jnp.float32),
    mesh=_mesh,
    scratch_types=[
        pltpu.VMEM((GRP, CH), jnp.int32),
        pltpu.VMEM((GRP, CH), jnp.int32),
        pltpu.VMEM((CH, D), jnp.float32),
        pltpu.VMEM((CH, D), jnp.float32),
        pltpu.VMEM((ZR, D), jnp.float32),
        pltpu.VMEM_SHARED((N, D), jnp.float32),
        pltpu.SemaphoreType.DMA,
        pltpu.SemaphoreType.DMA,
    ],
)
def _g(t_hbm, src_hbm, dst_hbm, out_hbm,
       src_g, dst_g, rows_a, rows_b, zero_v, acc, sem_a, sem_b):
    cid = lax.axis_index("c")
    sid = lax.axis_index("s")
    wid = cid * NS + sid

    def zfill(i, c):
        zero_v[i // 8, pl.ds((i % 8) * 16, 16)] = jnp.zeros((16,), jnp.float32)
        return c

    lax.fori_loop(0, ZR * 8, zfill, 0)

    def zcopy(i, c):
        pltpu.sync_copy(zero_v, acc.at[pl.ds(sid * DSTEP + i * ZR, ZR)])
        return c

    lax.fori_loop(0, DSIZE // ZR, zcopy, 0)
    plsc.subcore_barrier()

    def group(g, c):
        gb = g * GRP
        pltpu.sync_copy(src_hbm.at[wid, pl.ds(gb, GRP)], src_g)
        pltpu.sync_copy(dst_hbm.at[wid, pl.ds(gb, GRP)], dst_g)
        pltpu.async_copy(t_hbm.at[src_g.at[0]], rows_a, sem_a)

        def pair(p, c2):
            j = 2 * p
            pltpu.make_async_copy(t_hbm.at[src_g.at[j]], rows_a, sem_a).wait()
            pltpu.async_copy(t_hbm.at[src_g.at[j + 1]], rows_b, sem_b)
            pltpu.sync_copy(rows_a, acc.at[dst_g.at[j]], add=True)
            pltpu.make_async_copy(
                t_hbm.at[src_g.at[j + 1]], rows_b, sem_b).wait()

            @pl.when(p < GP - 1)
            def _():
                pltpu.async_copy(t_hbm.at[src_g.at[j + 2]], rows_a, sem_a)

            pltpu.sync_copy(rows_b, acc.at[dst_g.at[j + 1]], add=True)
            return c2

        lax.fori_loop(0, GP, pair, 0)
        return c

    lax.fori_loop(0, NGRP, group, 0)
    plsc.subcore_barrier()
    pltpu.sync_copy(acc.at[pl.ds(sid * DSTEP, DSIZE)],
                    out_hbm.at[cid, pl.ds(sid * DSTEP, DSIZE)])


@functools.partial(
    pl.kernel,
    out_type=jax.ShapeDtypeStruct((NC, N, D), jnp.float32),
    mesh=_mesh,
    scratch_types=[
        pltpu.VMEM((NCHUNK, CH), jnp.int32),
        pltpu.VMEM((CH, D), jnp.float32),
        pltpu.VMEM((ZR, D), jnp.float32),
        pltpu.VMEM_SHARED((N, D), jnp.float32),
    ],
)
def _deg(src_hbm, out_hbm, src_v, ones_v, zero_v, acc):
    cid = lax.axis_index("c")
    sid = lax.axis_index("s")
    wid = cid * NS + sid

    e0 = jnp.where(lax.iota(jnp.int32, 16) == 0, 1.0, 0.0).astype(jnp.float32)
    z16 = jnp.zeros((16,), jnp.float32)

    def fill(i, c):
        r = i // 8
        k = i % 8
        ones_v[r, pl.ds(k * 16, 16)] = jnp.where(k == 0, e0, z16)
        return c

    lax.fori_loop(0, CH * 8, fill, 0)

    def zfill(i, c):
        zero_v[i // 8, pl.ds((i % 8) * 16, 16)] = z16
        return c

    lax.fori_loop(0, ZR * 8, zfill, 0)

    def zcopy(i, c):
        pltpu.sync_copy(zero_v, acc.at[pl.ds(sid * DSTEP + i * ZR, ZR)])
        return c

    lax.fori_loop(0, DSIZE // ZR, zcopy, 0)
    plsc.subcore_barrier()

    pltpu.sync_copy(src_hbm.at[wid], src_v)

    def body(j, c):
        pltpu.sync_copy(ones_v, acc.at[src_v.at[j]], add=True)
        return c

    lax.fori_loop(0, NCHUNK, body, 0)
    plsc.subcore_barrier()
    pltpu.sync_copy(acc.at[pl.ds(sid * DSTEP, DSIZE)],
                    out_hbm.at[cid, pl.ds(sid * DSTEP, DSIZE)])


def _tc_scale_body(deg_ref, x_ref, dis_ref, xs_ref):
    d = deg_ref[0] + deg_ref[1]
    dis = jnp.where(d > 0.0, lax.rsqrt(jnp.where(d > 0.0, d, 1.0)), 0.0)
    dis_ref[...] = dis
    xs_ref[...] = x_ref[...] * dis


def _tc_scale(deg2, x):
    return pl.pallas_call(
        _tc_scale_body,
        grid=(GRID,),
        in_specs=[
            pl.BlockSpec((NC, RB, 1), lambda i: (0, i, 0)),
            pl.BlockSpec((RB, D), lambda i: (i, 0)),
        ],
        out_specs=[
            pl.BlockSpec((RB, 1), lambda i: (i, 0)),
            pl.BlockSpec((RB, D), lambda i: (i, 0)),
        ],
        out_shape=[
            jax.ShapeDtypeStruct((N, 1), jnp.float32),
            jax.ShapeDtypeStruct((N, D), jnp.float32),
        ],
    )(deg2, x)


def _tc_mid1_body(u_ref, dis_ref, t_ref, w_ref, outp_ref, y_ref):
    dis = dis_ref[...]
    tx1 = -(dis * (u_ref[0] + u_ref[1]))
    outp_ref[...] = (
        jnp.dot(t_ref[...], w_ref[0], preferred_element_type=jnp.float32)
        + jnp.dot(tx1, w_ref[1], preferred_element_type=jnp.float32))
    y_ref[...] = dis * tx1


def _tc_mid1(u2, dis, t, w):
    return pl.pallas_call(
        _tc_mid1_body,
        grid=(GRID,),
        in_specs=[
            pl.BlockSpec((NC, RB, D), lambda i: (0, i, 0)),
            pl.BlockSpec((RB, 1), lambda i: (i, 0)),
            pl.BlockSpec((RB, D), lambda i: (i, 0)),
            pl.BlockSpec((3, D, D), lambda i: (0, 0, 0)),
        ],
        out_specs=[
            pl.BlockSpec((RB, D), lambda i: (i, 0)),
            pl.BlockSpec((RB, D), lambda i: (i, 0)),
        ],
        out_shape=[
            jax.ShapeDtypeStruct((N, D), jnp.float32),
            jax.ShapeDtypeStruct((N, D), jnp.float32),
        ],
    )(u2, dis, t, w)


def _tc_mid2_body(u_ref, dis_ref, t_ref, outp_ref, w_ref, b_ref, h_ref, hs_ref):
    dis = dis_ref[...]
    tx2 = -2.0 * (dis * (u_ref[0] + u_ref[1])) - t_ref[...]
    h = jnp.maximum(
        outp_ref[...]
        + jnp.dot(tx2, w_ref[2], preferred_element_type=jnp.float32)
        + b_ref[...], 0.0)
    h_ref[...] = h
    hs_ref[...] = dis * h


def _tc_mid2(u2, dis, t, outp, w, b):
    return pl.pallas_call(
        _tc_mid2_body,
        grid=(GRID,),
        in_specs=[
            pl.BlockSpec((NC, RB, D), lambda i: (0, i, 0)),
            pl.BlockSpec((RB, 1), lambda i: (i, 0)),
            pl.BlockSpec((RB, D), lambda i: (i, 0)),
            pl.BlockSpec((RB, D), lambda i: (i, 0)),
            pl.BlockSpec((3, D, D), lambda i: (0, 0, 0)),
            pl.BlockSpec((D,), lambda i: (0,)),
        ],
        out_specs=[
            pl.BlockSpec((RB, D), lambda i: (i, 0)),
            pl.BlockSpec((RB, D), lambda i: (i, 0)),
        ],
        out_shape=[
            jax.ShapeDtypeStruct((N, D), jnp.float32),
            jax.ShapeDtypeStruct((N, D), jnp.float32),
        ],
    )(u2, dis, t, outp, w, b)


def _tc_final_body(u_ref, dis_ref, t_ref, outp_ref, w_ref, b_ref, o_ref):
    dis = dis_ref[...]
    tx2 = -2.0 * (dis * (u_ref[0] + u_ref[1])) - t_ref[...]
    o_ref[...] = jnp.maximum(
        outp_ref[...]
        + jnp.dot(tx2, w_ref[2], preferred_element_type=jnp.float32)
        + b_ref[...], 0.0)


def _tc_final(u2, dis, t, outp, w, b):
    return pl.pallas_call(
        _tc_final_body,
        grid=(GRID,),
        in_specs=[
            pl.BlockSpec((NC, RB, D), lambda i: (0, i, 0)),
            pl.BlockSpec((RB, 1), lambda i: (i, 0)),
            pl.BlockSpec((RB, D), lambda i: (i, 0)),
            pl.BlockSpec((RB, D), lambda i: (i, 0)),
            pl.BlockSpec((3, D, D), lambda i: (0, 0, 0)),
            pl.BlockSpec((D,), lambda i: (0,)),
        ],
        out_specs=pl.BlockSpec((RB, D), lambda i: (i, 0)),
        out_shape=jax.ShapeDtypeStruct((N, D), jnp.float32),
    )(u2, dis, t, outp, w, b)


def kernel(x, edge_index, W1, b1, W2, b2):
    src_r = edge_index[0].reshape(NW, NCHUNK, CH)
    dst_r = edge_index[1].reshape(NW, NCHUNK, CH)

    deg2 = jax.lax.slice(_deg(src_r), (0, 0, 0), (NC, N, 1))
    dis, xs = _tc_scale(deg2, x)

    u1 = _g(xs, src_r, dst_r)
    outp, y1 = _tc_mid1(u1, dis, x, W1)
    u2 = _g(y1, src_r, dst_r)
    h, hs = _tc_mid2(u2, dis, x, outp, W1, b1)

    u3 = _g(hs, src_r, dst_r)
    outp2, y2 = _tc_mid1(u3, dis, h, W2)
    u4 = _g(y2, src_r, dst_r)
    return _tc_final(u4, dis, h, outp2, W2, b2)


# speedup vs baseline: 1.1173x; 1.0025x over previous
import functools

import jax
import jax.numpy as jnp
from jax import lax
from jax.experimental import pallas as pl
from jax.experimental.pallas import tpu as pltpu
from jax.experimental.pallas import tpu_sc as plsc

N = 10000
E = 320000
D = 128

NC = 2
NS = 16
NW = NC * NS
EPW = E // NW
CH = 125
NCHUNK = EPW // CH
ZR = 40

DSTEP = 624
DSIZE = 640

RB = 1000
GRID = N // RB

_mesh = plsc.VectorSubcoreMesh(core_axis_name="c", subcore_axis_name="s")


GRP = 40
NGRP = NCHUNK // GRP
GP = GRP // 2


@functools.partial(
    pl.kernel,
    out_type=jax.ShapeDtypeStruct((NC, N, D), jnp.float32),
    mesh=_mesh,
    scratch_types=[
        pltpu.VMEM((GRP, CH), jnp.int32),
        pltpu.VMEM((GRP, CH), jnp.int32),
        pltpu.VMEM((CH, D), jnp.float32),
        pltpu.VMEM((CH, D), jnp.float32),
        pltpu.VMEM((ZR, D), jnp.float32),
        pltpu.VMEM_SHARED((N, D), jnp.float32),
        pltpu.SemaphoreType.DMA,
        pltpu.SemaphoreType.DMA,
    ],
)
def _g(t_hbm, src_hbm, dst_hbm, out_hbm,
       src_g, dst_g, rows_a, rows_b, zero_v, acc, sem_a, sem_b):
    cid = lax.axis_index("c")
    sid = lax.axis_index("s")
    wid = cid * NS + sid

    def zfill(i, c):
        zero_v[i // 8, pl.ds((i % 8) * 16, 16)] = jnp.zeros((16,), jnp.float32)
        return c

    lax.fori_loop(0, ZR * 8, zfill, 0)

    def zcopy(i, c):
        pltpu.sync_copy(zero_v, acc.at[pl.ds(sid * DSTEP + i * ZR, ZR)])
        return c

    lax.fori_loop(0, DSIZE // ZR, zcopy, 0)
    plsc.subcore_barrier()

    def group(g, c):
        gb = g * GRP
        pltpu.sync_copy(src_hbm.at[wid, pl.ds(gb, GRP)], src_g)
        pltpu.sync_copy(dst_hbm.at[wid, pl.ds(gb, GRP)], dst_g)
        pltpu.async_copy(t_hbm.at[src_g.at[0]], rows_a, sem_a)

        def pair(p, c2):
            j = 2 * p
            pltpu.make_async_copy(t_hbm.at[src_g.at[j]], rows_a, sem_a).wait()
            pltpu.async_copy(t_hbm.at[src_g.at[j + 1]], rows_b, sem_b)
            pltpu.sync_copy(rows_a, acc.at[dst_g.at[j]], add=True)
            pltpu.make_async_copy(
                t_hbm.at[src_g.at[j + 1]], rows_b, sem_b).wait()

            @pl.when(p < GP - 1)
            def _():
                pltpu.async_copy(t_hbm.at[src_g.at[j + 2]], rows_a, sem_a)

            pltpu.sync_copy(rows_b, acc.at[dst_g.at[j + 1]], add=True)
            return c2

        lax.fori_loop(0, GP, pair, 0)
        return c

    lax.fori_loop(0, NGRP, group, 0)
    plsc.subcore_barrier()
    pltpu.sync_copy(acc.at[pl.ds(sid * DSTEP, DSIZE)],
                    out_hbm.at[cid, pl.ds(sid * DSTEP, DSIZE)])


@functools.partial(
    pl.kernel,
    out_type=jax.ShapeDtypeStruct((NC, N, D), jnp.float32),
    mesh=_mesh,
    scratch_types=[
        pltpu.VMEM((NCHUNK, CH), jnp.int32),
        pltpu.VMEM((CH, D), jnp.float32),
        pltpu.VMEM((ZR, D), jnp.float32),
        pltpu.VMEM_SHARED((N, D), jnp.float32),
    ],
)
def _deg(src_hbm, out_hbm, src_v, ones_v, zero_v, acc):
    cid = lax.axis_index("c")
    sid = lax.axis_index("s")
    wid = cid * NS + sid

    e0 = jnp.where(lax.iota(jnp.int32, 16) == 0, 1.0, 0.0).astype(jnp.float32)
    z16 = jnp.zeros((16,), jnp.float32)

    def fill(i, c):
        r = i // 8
        k = i % 8
        ones_v[r, pl.ds(k * 16, 16)] = jnp.where(k == 0, e0, z16)
        return c

    lax.fori_loop(0, CH * 8, fill, 0)

    def zfill(i, c):
        zero_v[i // 8, pl.ds((i % 8) * 16, 16)] = z16
        return c

    lax.fori_loop(0, ZR * 8, zfill, 0)

    def zcopy(i, c):
        pltpu.sync_copy(zero_v, acc.at[pl.ds(sid * DSTEP + i * ZR, ZR)])
        return c

    lax.fori_loop(0, DSIZE // ZR, zcopy, 0)
    plsc.subcore_barrier()

    pltpu.sync_copy(src_hbm.at[wid], src_v)

    def body(j, c):
        pltpu.sync_copy(ones_v, acc.at[src_v.at[j]], add=True)
        return c

    lax.fori_loop(0, NCHUNK, body, 0)
    plsc.subcore_barrier()
    pltpu.sync_copy(acc.at[pl.ds(sid * DSTEP, DSIZE)],
                    out_hbm.at[cid, pl.ds(sid * DSTEP, DSIZE)])


def _tc_scale_body(deg_ref, x_ref, dis_ref, xs_ref):
    d = deg_ref[0] + deg_ref[1]
    dis = jnp.where(d > 0.0, lax.rsqrt(jnp.where(d > 0.0, d, 1.0)), 0.0)
    dis_ref[...] = dis
    xs_ref[...] = x_ref[...] * dis


def _tc_scale(deg2, x):
    return pl.pallas_call(
        _tc_scale_body,
        grid=(GRID,),
        in_specs=[
            pl.BlockSpec((NC, RB, 1), lambda i: (0, i, 0)),
            pl.BlockSpec((RB, D), lambda i: (i, 0)),
        ],
        out_specs=[
            pl.BlockSpec((RB, 1), lambda i: (i, 0)),
            pl.BlockSpec((RB, D), lambda i: (i, 0)),
        ],
        out_shape=[
            jax.ShapeDtypeStruct((N, 1), jnp.float32),
            jax.ShapeDtypeStruct((N, D), jnp.float32),
        ],
    )(deg2, x)


def _tc_y_body(u_ref, dis_ref, y_ref):
    dis = dis_ref[...]
    y_ref[...] = -(dis * dis * (u_ref[0] + u_ref[1]))


def _tc_y(u2, dis):
    return pl.pallas_call(
        _tc_y_body,
        grid=(GRID,),
        in_specs=[
            pl.BlockSpec((NC, RB, D), lambda i: (0, i, 0)),
            pl.BlockSpec((RB, 1), lambda i: (i, 0)),
        ],
        out_specs=pl.BlockSpec((RB, D), lambda i: (i, 0)),
        out_shape=jax.ShapeDtypeStruct((N, D), jnp.float32),
    )(u2, dis)


def _tc_outp_body(u_ref, dis_ref, t_ref, w_ref, outp_ref):
    dis = dis_ref[...]
    tx1 = -(dis * (u_ref[0] + u_ref[1]))
    outp_ref[...] = (
        jnp.dot(t_ref[...], w_ref[0], preferred_element_type=jnp.float32)
        + jnp.dot(tx1, w_ref[1], preferred_element_type=jnp.float32))


def _tc_outp(u2, dis, t, w):
    return pl.pallas_call(
        _tc_outp_body,
        grid=(GRID,),
        in_specs=[
            pl.BlockSpec((NC, RB, D), lambda i: (0, i, 0)),
            pl.BlockSpec((RB, 1), lambda i: (i, 0)),
            pl.BlockSpec((RB, D), lambda i: (i, 0)),
            pl.BlockSpec((3, D, D), lambda i: (0, 0, 0)),
        ],
        out_specs=pl.BlockSpec((RB, D), lambda i: (i, 0)),
        out_shape=jax.ShapeDtypeStruct((N, D), jnp.float32),
    )(u2, dis, t, w)


def _tc_mid2_body(u_ref, dis_ref, t_ref, outp_ref, w_ref, b_ref, h_ref, hs_ref):
    dis = dis_ref[...]
    tx2 = -2.0 * (dis * (u_ref[0] + u_ref[1])) - t_ref[...]
    h = jnp.maximum(
        outp_ref[...]
        + jnp.dot(tx2, w_ref[2], preferred_element_type=jnp.float32)
        + b_ref[...], 0.0)
    h_ref[...] = h
    hs_ref[...] = dis * h


def _tc_mid2(u2, dis, t, outp, w, b):
    return pl.pallas_call(
        _tc_mid2_body,
        grid=(GRID,),
        in_specs=[
            pl.BlockSpec((NC, RB, D), lambda i: (0, i, 0)),
            pl.BlockSpec((RB, 1), lambda i: (i, 0)),
            pl.BlockSpec((RB, D), lambda i: (i, 0)),
            pl.BlockSpec((RB, D), lambda i: (i, 0)),
            pl.BlockSpec((3, D, D), lambda i: (0, 0, 0)),
            pl.BlockSpec((D,), lambda i: (0,)),
        ],
        out_specs=[
            pl.BlockSpec((RB, D), lambda i: (i, 0)),
            pl.BlockSpec((RB, D), lambda i: (i, 0)),
        ],
        out_shape=[
            jax.ShapeDtypeStruct((N, D), jnp.float32),
            jax.ShapeDtypeStruct((N, D), jnp.float32),
        ],
    )(u2, dis, t, outp, w, b)


def _tc_final_body(u_ref, dis_ref, t_ref, outp_ref, w_ref, b_ref, o_ref):
    dis = dis_ref[...]
    tx2 = -2.0 * (dis * (u_ref[0] + u_ref[1])) - t_ref[...]
    o_ref[...] = jnp.maximum(
        outp_ref[...]
        + jnp.dot(tx2, w_ref[2], preferred_element_type=jnp.float32)
        + b_ref[...], 0.0)


def _tc_final(u2, dis, t, outp, w, b):
    return pl.pallas_call(
        _tc_final_body,
        grid=(GRID,),
        in_specs=[
            pl.BlockSpec((NC, RB, D), lambda i: (0, i, 0)),
            pl.BlockSpec((RB, 1), lambda i: (i, 0)),
            pl.BlockSpec((RB, D), lambda i: (i, 0)),
            pl.BlockSpec((RB, D), lambda i: (i, 0)),
            pl.BlockSpec((3, D, D), lambda i: (0, 0, 0)),
            pl.BlockSpec((D,), lambda i: (0,)),
        ],
        out_specs=pl.BlockSpec((RB, D), lambda i: (i, 0)),
        out_shape=jax.ShapeDtypeStruct((N, D), jnp.float32),
    )(u2, dis, t, outp, w, b)


def kernel(x, edge_index, W1, b1, W2, b2):
    src_r = edge_index[0].reshape(NW, NCHUNK, CH)
    dst_r = edge_index[1].reshape(NW, NCHUNK, CH)

    deg2 = jax.lax.slice(_deg(src_r), (0, 0, 0), (NC, N, 1))
    dis, xs = _tc_scale(deg2, x)

    u1 = _g(xs, src_r, dst_r)
    y1 = _tc_y(u1, dis)
    u2 = _g(y1, src_r, dst_r)
    outp = _tc_outp(u1, dis, x, W1)
    h, hs = _tc_mid2(u2, dis, x, outp, W1, b1)

    u3 = _g(hs, src_r, dst_r)
    y2 = _tc_y(u3, dis)
    u4 = _g(y2, src_r, dst_r)
    outp2 = _tc_outp(u3, dis, h, W2)
    return _tc_final(u4, dis, h, outp2, W2, b2)


# speedup vs baseline: 1.1327x; 1.0138x over previous
import functools

import jax
import jax.numpy as jnp
from jax import lax
from jax.experimental import pallas as pl
from jax.experimental.pallas import tpu as pltpu
from jax.experimental.pallas import tpu_sc as plsc

N = 10000
E = 320000
D = 128

NC = 2
NS = 16
NW = NC * NS
EPW = E // NW
CH = 125
NCHUNK = EPW // CH
ZR = 40

DSTEP = 624
DSIZE = 640

RB = 2000
GRID = N // RB

_mesh = plsc.VectorSubcoreMesh(core_axis_name="c", subcore_axis_name="s")


GRP = 40
NGRP = NCHUNK // GRP
GP = GRP // 2


@functools.partial(
    pl.kernel,
    out_type=jax.ShapeDtypeStruct((NC, N, D), jnp.float32),
    mesh=_mesh,
    scratch_types=[
        pltpu.VMEM((GRP, CH), jnp.int32),
        pltpu.VMEM((GRP, CH), jnp.int32),
        pltpu.VMEM((CH, D), jnp.float32),
        pltpu.VMEM((CH, D), jnp.float32),
        pltpu.VMEM((ZR, D), jnp.float32),
        pltpu.VMEM_SHARED((N, D), jnp.float32),
        pltpu.SemaphoreType.DMA,
        pltpu.SemaphoreType.DMA,
    ],
)
def _g(t_hbm, src_hbm, dst_hbm, out_hbm,
       src_g, dst_g, rows_a, rows_b, zero_v, acc, sem_a, sem_b):
    cid = lax.axis_index("c")
    sid = lax.axis_index("s")
    wid = cid * NS + sid

    def zfill(i, c):
        zero_v[i // 8, pl.ds((i % 8) * 16, 16)] = jnp.zeros((16,), jnp.float32)
        return c

    lax.fori_loop(0, ZR * 8, zfill, 0)

    def zcopy(i, c):
        pltpu.sync_copy(zero_v, acc.at[pl.ds(sid * DSTEP + i * ZR, ZR)])
        return c

    lax.fori_loop(0, DSIZE // ZR, zcopy, 0)
    plsc.subcore_barrier()

    def group(g, c):
        gb = g * GRP
        pltpu.sync_copy(src_hbm.at[wid, pl.ds(gb, GRP)], src_g)
        pltpu.sync_copy(dst_hbm.at[wid, pl.ds(gb, GRP)], dst_g)
        pltpu.async_copy(t_hbm.at[src_g.at[0]], rows_a, sem_a)

        def pair(p, c2):
            j = 2 * p
            pltpu.make_async_copy(t_hbm.at[src_g.at[j]], rows_a, sem_a).wait()
            pltpu.async_copy(t_hbm.at[src_g.at[j + 1]], rows_b, sem_b)
            pltpu.sync_copy(rows_a, acc.at[dst_g.at[j]], add=True)
            pltpu.make_async_copy(
                t_hbm.at[src_g.at[j + 1]], rows_b, sem_b).wait()

            @pl.when(p < GP - 1)
            def _():
                pltpu.async_copy(t_hbm.at[src_g.at[j + 2]], rows_a, sem_a)

            pltpu.sync_copy(rows_b, acc.at[dst_g.at[j + 1]], add=True)
            return c2

        lax.fori_loop(0, GP, pair, 0)
        return c

    lax.fori_loop(0, NGRP, group, 0)
    plsc.subcore_barrier()
    pltpu.sync_copy(acc.at[pl.ds(sid * DSTEP, DSIZE)],
                    out_hbm.at[cid, pl.ds(sid * DSTEP, DSIZE)])


@functools.partial(
    pl.kernel,
    out_type=jax.ShapeDtypeStruct((NC, N, D), jnp.float32),
    mesh=_mesh,
    scratch_types=[
        pltpu.VMEM((NCHUNK, CH), jnp.int32),
        pltpu.VMEM((CH, D), jnp.float32),
        pltpu.VMEM((ZR, D), jnp.float32),
        pltpu.VMEM_SHARED((N, D), jnp.float32),
    ],
)
def _deg(src_hbm, out_hbm, src_v, ones_v, zero_v, acc):
    cid = lax.axis_index("c")
    sid = lax.axis_index("s")
    wid = cid * NS + sid

    e0 = jnp.where(lax.iota(jnp.int32, 16) == 0, 1.0, 0.0).astype(jnp.float32)
    z16 = jnp.zeros((16,), jnp.float32)

    def fill(i, c):
        r = i // 8
        k = i % 8
        ones_v[r, pl.ds(k * 16, 16)] = jnp.where(k == 0, e0, z16)
        return c

    lax.fori_loop(0, CH * 8, fill, 0)

    def zfill(i, c):
        zero_v[i // 8, pl.ds((i % 8) * 16, 16)] = z16
        return c

    lax.fori_loop(0, ZR * 8, zfill, 0)

    def zcopy(i, c):
        pltpu.sync_copy(zero_v, acc.at[pl.ds(sid * DSTEP + i * ZR, ZR)])
        return c

    lax.fori_loop(0, DSIZE // ZR, zcopy, 0)
    plsc.subcore_barrier()

    pltpu.sync_copy(src_hbm.at[wid], src_v)

    def body(j, c):
        pltpu.sync_copy(ones_v, acc.at[src_v.at[j]], add=True)
        return c

    lax.fori_loop(0, NCHUNK, body, 0)
    plsc.subcore_barrier()
    pltpu.sync_copy(acc.at[pl.ds(sid * DSTEP, DSIZE)],
                    out_hbm.at[cid, pl.ds(sid * DSTEP, DSIZE)])


def _tc_scale_body(deg_ref, x_ref, dis_ref, xs_ref):
    d = deg_ref[0] + deg_ref[1]
    dis = jnp.where(d > 0.0, lax.rsqrt(jnp.where(d > 0.0, d, 1.0)), 0.0)
    dis_ref[...] = dis
    xs_ref[...] = x_ref[...] * dis


def _tc_scale(deg2, x):
    return pl.pallas_call(
        _tc_scale_body,
        grid=(GRID,),
        in_specs=[
            pl.BlockSpec((NC, RB, 1), lambda i: (0, i, 0)),
            pl.BlockSpec((RB, D), lambda i: (i, 0)),
        ],
        out_specs=[
            pl.BlockSpec((RB, 1), lambda i: (i, 0)),
            pl.BlockSpec((RB, D), lambda i: (i, 0)),
        ],
        out_shape=[
            jax.ShapeDtypeStruct((N, 1), jnp.float32),
            jax.ShapeDtypeStruct((N, D), jnp.float32),
        ],
    )(deg2, x)


def _tc_y_body(u_ref, dis_ref, y_ref):
    dis = dis_ref[...]
    y_ref[...] = -(dis * dis * (u_ref[0] + u_ref[1]))


def _tc_y(u2, dis):
    return pl.pallas_call(
        _tc_y_body,
        grid=(GRID,),
        in_specs=[
            pl.BlockSpec((NC, RB, D), lambda i: (0, i, 0)),
            pl.BlockSpec((RB, 1), lambda i: (i, 0)),
        ],
        out_specs=pl.BlockSpec((RB, D), lambda i: (i, 0)),
        out_shape=jax.ShapeDtypeStruct((N, D), jnp.float32),
    )(u2, dis)


def _tc_outp_body(u_ref, dis_ref, t_ref, w_ref, outp_ref):
    dis = dis_ref[...]
    tx1 = -(dis * (u_ref[0] + u_ref[1]))
    outp_ref[...] = (
        jnp.dot(t_ref[...], w_ref[0], preferred_element_type=jnp.float32)
        + jnp.dot(tx1, w_ref[1], preferred_element_type=jnp.float32))


def _tc_outp(u2, dis, t, w):
    return pl.pallas_call(
        _tc_outp_body,
        grid=(GRID,),
        in_specs=[
            pl.BlockSpec((NC, RB, D), lambda i: (0, i, 0)),
            pl.BlockSpec((RB, 1), lambda i: (i, 0)),
            pl.BlockSpec((RB, D), lambda i: (i, 0)),
            pl.BlockSpec((3, D, D), lambda i: (0, 0, 0)),
        ],
        out_specs=pl.BlockSpec((RB, D), lambda i: (i, 0)),
        out_shape=jax.ShapeDtypeStruct((N, D), jnp.float32),
    )(u2, dis, t, w)


def _tc_mid2_body(u_ref, dis_ref, t_ref, outp_ref, w_ref, b_ref, h_ref, hs_ref):
    dis = dis_ref[...]
    tx2 = -2.0 * (dis * (u_ref[0] + u_ref[1])) - t_ref[...]
    h = jnp.maximum(
        outp_ref[...]
        + jnp.dot(tx2, w_ref[2], preferred_element_type=jnp.float32)
        + b_ref[...], 0.0)
    h_ref[...] = h
    hs_ref[...] = dis * h


def _tc_mid2(u2, dis, t, outp, w, b):
    return pl.pallas_call(
        _tc_mid2_body,
        grid=(GRID,),
        in_specs=[
            pl.BlockSpec((NC, RB, D), lambda i: (0, i, 0)),
            pl.BlockSpec((RB, 1), lambda i: (i, 0)),
            pl.BlockSpec((RB, D), lambda i: (i, 0)),
            pl.BlockSpec((RB, D), lambda i: (i, 0)),
            pl.BlockSpec((3, D, D), lambda i: (0, 0, 0)),
            pl.BlockSpec((D,), lambda i: (0,)),
        ],
        out_specs=[
            pl.BlockSpec((RB, D), lambda i: (i, 0)),
            pl.BlockSpec((RB, D), lambda i: (i, 0)),
        ],
        out_shape=[
            jax.ShapeDtypeStruct((N, D), jnp.float32),
            jax.ShapeDtypeStruct((N, D), jnp.float32),
        ],
    )(u2, dis, t, outp, w, b)


def _tc_final_body(u_ref, dis_ref, t_ref, outp_ref, w_ref, b_ref, o_ref):
    dis = dis_ref[...]
    tx2 = -2.0 * (dis * (u_ref[0] + u_ref[1])) - t_ref[...]
    o_ref[...] = jnp.maximum(
        outp_ref[...]
        + jnp.dot(tx2, w_ref[2], preferred_element_type=jnp.float32)
        + b_ref[...], 0.0)


def _tc_final(u2, dis, t, outp, w, b):
    return pl.pallas_call(
        _tc_final_body,
        grid=(GRID,),
        in_specs=[
            pl.BlockSpec((NC, RB, D), lambda i: (0, i, 0)),
            pl.BlockSpec((RB, 1), lambda i: (i, 0)),
            pl.BlockSpec((RB, D), lambda i: (i, 0)),
            pl.BlockSpec((RB, D), lambda i: (i, 0)),
            pl.BlockSpec((3, D, D), lambda i: (0, 0, 0)),
            pl.BlockSpec((D,), lambda i: (0,)),
        ],
        out_specs=pl.BlockSpec((RB, D), lambda i: (i, 0)),
        out_shape=jax.ShapeDtypeStruct((N, D), jnp.float32),
    )(u2, dis, t, outp, w, b)


def kernel(x, edge_index, W1, b1, W2, b2):
    src_r = edge_index[0].reshape(NW, NCHUNK, CH)
    dst_r = edge_index[1].reshape(NW, NCHUNK, CH)

    deg2 = jax.lax.slice(_deg(src_r), (0, 0, 0), (NC, N, 1))
    dis, xs = _tc_scale(deg2, x)

    u1 = _g(xs, src_r, dst_r)
    y1 = _tc_y(u1, dis)
    u2 = _g(y1, src_r, dst_r)
    outp = _tc_outp(u1, dis, x, W1)
    h, hs = _tc_mid2(u2, dis, x, outp, W1, b1)

    u3 = _g(hs, src_r, dst_r)
    y2 = _tc_y(u3, dis)
    u4 = _g(y2, src_r, dst_r)
    outp2 = _tc_outp(u3, dis, h, W2)
    return _tc_final(u4, dis, h, outp2, W2, b2)


# speedup vs baseline: 1.1453x; 1.0112x over previous
import functools

import jax
import jax.numpy as jnp
from jax import lax
from jax.experimental import pallas as pl
from jax.experimental.pallas import tpu as pltpu
from jax.experimental.pallas import tpu_sc as plsc

N = 10000
E = 320000
D = 128

NC = 2
NS = 16
NW = NC * NS
EPW = E // NW
CH = 125
NCHUNK = EPW // CH
ZR = 40

DSTEP = 624
DSIZE = 640

RB = 5000
GRID = N // RB

_mesh = plsc.VectorSubcoreMesh(core_axis_name="c", subcore_axis_name="s")


GRP = 40
NGRP = NCHUNK // GRP
GP = GRP // 2


@functools.partial(
    pl.kernel,
    out_type=jax.ShapeDtypeStruct((NC, N, D), jnp.float32),
    mesh=_mesh,
    scratch_types=[
        pltpu.VMEM((GRP, CH), jnp.int32),
        pltpu.VMEM((GRP, CH), jnp.int32),
        pltpu.VMEM((CH, D), jnp.float32),
        pltpu.VMEM((CH, D), jnp.float32),
        pltpu.VMEM((ZR, D), jnp.float32),
        pltpu.VMEM_SHARED((N, D), jnp.float32),
        pltpu.SemaphoreType.DMA,
        pltpu.SemaphoreType.DMA,
    ],
)
def _g(t_hbm, src_hbm, dst_hbm, out_hbm,
       src_g, dst_g, rows_a, rows_b, zero_v, acc, sem_a, sem_b):
    cid = lax.axis_index("c")
    sid = lax.axis_index("s")
    wid = cid * NS + sid

    def zfill(i, c):
        zero_v[i // 8, pl.ds((i % 8) * 16, 16)] = jnp.zeros((16,), jnp.float32)
        return c

    lax.fori_loop(0, ZR * 8, zfill, 0)

    def zcopy(i, c):
        pltpu.sync_copy(zero_v, acc.at[pl.ds(sid * DSTEP + i * ZR, ZR)])
        return c

    lax.fori_loop(0, DSIZE // ZR, zcopy, 0)
    plsc.subcore_barrier()

    def group(g, c):
        gb = g * GRP
        pltpu.sync_copy(src_hbm.at[wid, pl.ds(gb, GRP)], src_g)
        pltpu.sync_copy(dst_hbm.at[wid, pl.ds(gb, GRP)], dst_g)
        pltpu.async_copy(t_hbm.at[src_g.at[0]], rows_a, sem_a)

        def pair(p, c2):
            j = 2 * p
            pltpu.make_async_copy(t_hbm.at[src_g.at[j]], rows_a, sem_a).wait()
            pltpu.async_copy(t_hbm.at[src_g.at[j + 1]], rows_b, sem_b)
            pltpu.sync_copy(rows_a, acc.at[dst_g.at[j]], add=True)
            pltpu.make_async_copy(
                t_hbm.at[src_g.at[j + 1]], rows_b, sem_b).wait()

            @pl.when(p < GP - 1)
            def _():
                pltpu.async_copy(t_hbm.at[src_g.at[j + 2]], rows_a, sem_a)

            pltpu.sync_copy(rows_b, acc.at[dst_g.at[j + 1]], add=True)
            return c2

        lax.fori_loop(0, GP, pair, 0)
        return c

    lax.fori_loop(0, NGRP, group, 0)
    plsc.subcore_barrier()
    pltpu.sync_copy(acc.at[pl.ds(sid * DSTEP, DSIZE)],
                    out_hbm.at[cid, pl.ds(sid * DSTEP, DSIZE)])


@functools.partial(
    pl.kernel,
    out_type=jax.ShapeDtypeStruct((NC, N, D), jnp.float32),
    mesh=_mesh,
    scratch_types=[
        pltpu.VMEM((NCHUNK, CH), jnp.int32),
        pltpu.VMEM((CH, D), jnp.float32),
        pltpu.VMEM((ZR, D), jnp.float32),
        pltpu.VMEM_SHARED((N, D), jnp.float32),
    ],
)
def _deg(src_hbm, out_hbm, src_v, ones_v, zero_v, acc):
    cid = lax.axis_index("c")
    sid = lax.axis_index("s")
    wid = cid * NS + sid

    e0 = jnp.where(lax.iota(jnp.int32, 16) == 0, 1.0, 0.0).astype(jnp.float32)
    z16 = jnp.zeros((16,), jnp.float32)

    def fill(i, c):
        r = i // 8
        k = i % 8
        ones_v[r, pl.ds(k * 16, 16)] = jnp.where(k == 0, e0, z16)
        return c

    lax.fori_loop(0, CH * 8, fill, 0)

    def zfill(i, c):
        zero_v[i // 8, pl.ds((i % 8) * 16, 16)] = z16
        return c

    lax.fori_loop(0, ZR * 8, zfill, 0)

    def zcopy(i, c):
        pltpu.sync_copy(zero_v, acc.at[pl.ds(sid * DSTEP + i * ZR, ZR)])
        return c

    lax.fori_loop(0, DSIZE // ZR, zcopy, 0)
    plsc.subcore_barrier()

    pltpu.sync_copy(src_hbm.at[wid], src_v)

    def body(j, c):
        pltpu.sync_copy(ones_v, acc.at[src_v.at[j]], add=True)
        return c

    lax.fori_loop(0, NCHUNK, body, 0)
    plsc.subcore_barrier()
    pltpu.sync_copy(acc.at[pl.ds(sid * DSTEP, DSIZE)],
                    out_hbm.at[cid, pl.ds(sid * DSTEP, DSIZE)])


def _tc_scale_body(deg_ref, x_ref, dis_ref, xs_ref):
    d = deg_ref[0] + deg_ref[1]
    dis = jnp.where(d > 0.0, lax.rsqrt(jnp.where(d > 0.0, d, 1.0)), 0.0)
    dis_ref[...] = dis
    xs_ref[...] = x_ref[...] * dis


def _tc_scale(deg2, x):
    return pl.pallas_call(
        _tc_scale_body,
        grid=(GRID,),
        in_specs=[
            pl.BlockSpec((NC, RB, 1), lambda i: (0, i, 0)),
            pl.BlockSpec((RB, D), lambda i: (i, 0)),
        ],
        out_specs=[
            pl.BlockSpec((RB, 1), lambda i: (i, 0)),
            pl.BlockSpec((RB, D), lambda i: (i, 0)),
        ],
        out_shape=[
            jax.ShapeDtypeStruct((N, 1), jnp.float32),
            jax.ShapeDtypeStruct((N, D), jnp.float32),
        ],
    )(deg2, x)


def _tc_y_body(u_ref, dis_ref, y_ref):
    dis = dis_ref[...]
    y_ref[...] = -(dis * dis * (u_ref[0] + u_ref[1]))


def _tc_y(u2, dis):
    return pl.pallas_call(
        _tc_y_body,
        grid=(GRID,),
        in_specs=[
            pl.BlockSpec((NC, RB, D), lambda i: (0, i, 0)),
            pl.BlockSpec((RB, 1), lambda i: (i, 0)),
        ],
        out_specs=pl.BlockSpec((RB, D), lambda i: (i, 0)),
        out_shape=jax.ShapeDtypeStruct((N, D), jnp.float32),
    )(u2, dis)


def _tc_outp_body(u_ref, dis_ref, t_ref, w_ref, outp_ref):
    dis = dis_ref[...]
    tx1 = -(dis * (u_ref[0] + u_ref[1]))
    outp_ref[...] = (
        jnp.dot(t_ref[...], w_ref[0], preferred_element_type=jnp.float32)
        + jnp.dot(tx1, w_ref[1], preferred_element_type=jnp.float32))


def _tc_outp(u2, dis, t, w):
    return pl.pallas_call(
        _tc_outp_body,
        grid=(GRID,),
        in_specs=[
            pl.BlockSpec((NC, RB, D), lambda i: (0, i, 0)),
            pl.BlockSpec((RB, 1), lambda i: (i, 0)),
            pl.BlockSpec((RB, D), lambda i: (i, 0)),
            pl.BlockSpec((3, D, D), lambda i: (0, 0, 0)),
        ],
        out_specs=pl.BlockSpec((RB, D), lambda i: (i, 0)),
        out_shape=jax.ShapeDtypeStruct((N, D), jnp.float32),
    )(u2, dis, t, w)


def _tc_mid2_body(u_ref, dis_ref, t_ref, outp_ref, w_ref, b_ref, h_ref, hs_ref):
    dis = dis_ref[...]
    tx2 = -2.0 * (dis * (u_ref[0] + u_ref[1])) - t_ref[...]
    h = jnp.maximum(
        outp_ref[...]
        + jnp.dot(tx2, w_ref[2], preferred_element_type=jnp.float32)
        + b_ref[...], 0.0)
    h_ref[...] = h
    hs_ref[...] = dis * h


def _tc_mid2(u2, dis, t, outp, w, b):
    return pl.pallas_call(
        _tc_mid2_body,
        grid=(GRID,),
        in_specs=[
            pl.BlockSpec((NC, RB, D), lambda i: (0, i, 0)),
            pl.BlockSpec((RB, 1), lambda i: (i, 0)),
            pl.BlockSpec((RB, D), lambda i: (i, 0)),
            pl.BlockSpec((RB, D), lambda i: (i, 0)),
            pl.BlockSpec((3, D, D), lambda i: (0, 0, 0)),
            pl.BlockSpec((D,), lambda i: (0,)),
        ],
        out_specs=[
            pl.BlockSpec((RB, D), lambda i: (i, 0)),
            pl.BlockSpec((RB, D), lambda i: (i, 0)),
        ],
        out_shape=[
            jax.ShapeDtypeStruct((N, D), jnp.float32),
            jax.ShapeDtypeStruct((N, D), jnp.float32),
        ],
    )(u2, dis, t, outp, w, b)


def _tc_final_body(u_ref, dis_ref, t_ref, outp_ref, w_ref, b_ref, o_ref):
    dis = dis_ref[...]
    tx2 = -2.0 * (dis * (u_ref[0] + u_ref[1])) - t_ref[...]
    o_ref[...] = jnp.maximum(
        outp_ref[...]
        + jnp.dot(tx2, w_ref[2], preferred_element_type=jnp.float32)
        + b_ref[...], 0.0)


def _tc_final(u2, dis, t, outp, w, b):
    return pl.pallas_call(
        _tc_final_body,
        grid=(GRID,),
        in_specs=[
            pl.BlockSpec((NC, RB, D), lambda i: (0, i, 0)),
            pl.BlockSpec((RB, 1), lambda i: (i, 0)),
            pl.BlockSpec((RB, D), lambda i: (i, 0)),
            pl.BlockSpec((RB, D), lambda i: (i, 0)),
            pl.BlockSpec((3, D, D), lambda i: (0, 0, 0)),
            pl.BlockSpec((D,), lambda i: (0,)),
        ],
        out_specs=pl.BlockSpec((RB, D), lambda i: (i, 0)),
        out_shape=jax.ShapeDtypeStruct((N, D), jnp.float32),
    )(u2, dis, t, outp, w, b)


def kernel(x, edge_index, W1, b1, W2, b2):
    src_r = edge_index[0].reshape(NW, NCHUNK, CH)
    dst_r = edge_index[1].reshape(NW, NCHUNK, CH)

    deg2 = jax.lax.slice(_deg(src_r), (0, 0, 0), (NC, N, 1))
    dis, xs = _tc_scale(deg2, x)

    u1 = _g(xs, src_r, dst_r)
    y1 = _tc_y(u1, dis)
    u2 = _g(y1, src_r, dst_r)
    outp = _tc_outp(u1, dis, x, W1)
    h, hs = _tc_mid2(u2, dis, x, outp, W1, b1)

    u3 = _g(hs, src_r, dst_r)
    y2 = _tc_y(u3, dis)
    u4 = _g(y2, src_r, dst_r)
    outp2 = _tc_outp(u3, dis, h, W2)
    return _tc_final(u4, dis, h, outp2, W2, b2)
